# Initial kernel scaffold; baseline (speedup 1.0000x reference)
import jax
import jax.numpy as jnp
from jax.experimental import pallas as pl


def kernel(features, src_nodes, dstsrc2src_l1, dstsrc2dst_l1, dif_mat_l1, dstsrc2src_l2, dstsrc2dst_l2, dif_mat_l2, W1, W2, Wc):
    raise NotImplementedError("write your pallas kernel here")


# speedup vs baseline: 1.6255x; 1.6255x over previous
import functools

import jax
import jax.numpy as jnp
from jax import lax
from jax.experimental import pallas as pl
from jax.experimental.pallas import tpu as pltpu
from jax.experimental.pallas import tpu_sc as plsc

N_NODES, D_FEAT = 100000, 128
N0, N1, B = 30976, 2816, 256
INTERNAL, NUM_CLASSES = 128, 64

NC, NS = 2, 16
NW = NC * NS
L = 16
CHUNK = 128
N0_CHUNKS = N0 // CHUNK
N1_CHUNKS = N1 // CHUNK
B_CHUNKS = B // CHUNK
SRC_ROUNDS = -(-N0_CHUNKS // NW)


def _sc_gather_l1_body(features, src_nodes, d2s, d2d, src_out, dst_out,
                       j_v, idx_v, rows_v, sem, sem2):
    wid = lax.axis_index("s") * NC + lax.axis_index("c")

    def do_chunk(cid, idx_hbm, out_hbm):
        base = pl.multiple_of(cid * CHUNK, CHUNK)
        pltpu.sync_copy(idx_hbm.at[pl.ds(base, CHUNK)], j_v)
        pltpu.async_copy(src_nodes.at[j_v], idx_v, sem2).wait()
        pltpu.async_copy(features.at[idx_v], rows_v, sem).wait()
        pltpu.sync_copy(rows_v, out_hbm.at[pl.ds(base, CHUNK)])

    for t in range(SRC_ROUNDS):
        cid = wid + t * NW

        @pl.when(cid < N0_CHUNKS)
        def _():
            do_chunk(cid, d2s, src_out)

    @pl.when(wid < N1_CHUNKS)
    def _():
        do_chunk(wid, d2d, dst_out)


def _sc_gather_l1(features, src_nodes, d2s, d2d):
    mesh = plsc.VectorSubcoreMesh(core_axis_name="c", subcore_axis_name="s")
    return pl.kernel(
        _sc_gather_l1_body,
        out_type=[
            jax.ShapeDtypeStruct((N0, D_FEAT), jnp.float32),
            jax.ShapeDtypeStruct((N1, D_FEAT), jnp.float32),
        ],
        mesh=mesh,
        scratch_types=[
            pltpu.VMEM((CHUNK,), jnp.int32),
            pltpu.VMEM((CHUNK,), jnp.int32),
            pltpu.VMEM((CHUNK, D_FEAT), jnp.float32),
            pltpu.SemaphoreType.DMA,
            pltpu.SemaphoreType.DMA,
        ],
    )(features, src_nodes, d2s, d2d)


def _sc_gather_l2_body(h1, d2s, d2d, src_out, dst_out, j_v, rows_v, sem):
    wid = lax.axis_index("s") * NC + lax.axis_index("c")

    def do_chunk(cid, idx_hbm, out_hbm):
        base = pl.multiple_of(cid * CHUNK, CHUNK)
        pltpu.sync_copy(idx_hbm.at[pl.ds(base, CHUNK)], j_v)
        pltpu.async_copy(h1.at[j_v], rows_v, sem).wait()
        pltpu.sync_copy(rows_v, out_hbm.at[pl.ds(base, CHUNK)])

    @pl.when(wid < N1_CHUNKS)
    def _():
        do_chunk(wid, d2s, src_out)

    @pl.when((wid >= N1_CHUNKS) & (wid < N1_CHUNKS + B_CHUNKS))
    def _():
        do_chunk(wid - N1_CHUNKS, d2d, dst_out)


def _sc_gather_l2(h1, d2s, d2d):
    mesh = plsc.VectorSubcoreMesh(core_axis_name="c", subcore_axis_name="s")
    return pl.kernel(
        _sc_gather_l2_body,
        out_type=[
            jax.ShapeDtypeStruct((N1, INTERNAL), jnp.float32),
            jax.ShapeDtypeStruct((B, INTERNAL), jnp.float32),
        ],
        mesh=mesh,
        scratch_types=[
            pltpu.VMEM((CHUNK,), jnp.int32),
            pltpu.VMEM((CHUNK, INTERNAL), jnp.float32),
            pltpu.SemaphoreType.DMA,
        ],
    )(h1, d2s, d2d)


K_BLK = 1408
K_STEPS = N0 // K_BLK


def _tc_layer1_body(dm_ref, sf_ref, df_ref, w1_ref, out_ref, acc_ref):
    k = pl.program_id(0)

    @pl.when(k == 0)
    def _():
        acc_ref[...] = jnp.zeros_like(acc_ref)

    acc_ref[...] += jnp.dot(dm_ref[...], sf_ref[...],
                            preferred_element_type=jnp.float32)

    @pl.when(k == K_STEPS - 1)
    def _():
        w1 = w1_ref[...]
        h = (jnp.dot(acc_ref[...], w1[:D_FEAT, :],
                     preferred_element_type=jnp.float32)
             + jnp.dot(df_ref[...], w1[D_FEAT:, :],
                       preferred_element_type=jnp.float32))
        out_ref[...] = jnp.maximum(h, 0.0)


def _tc_layer1(dm1, src_feat, dst_feat, W1):
    return pl.pallas_call(
        _tc_layer1_body,
        grid=(K_STEPS,),
        in_specs=[
            pl.BlockSpec((N1, K_BLK), lambda k: (0, k)),
            pl.BlockSpec((K_BLK, D_FEAT), lambda k: (k, 0)),
            pl.BlockSpec((N1, D_FEAT), lambda k: (0, 0)),
            pl.BlockSpec((2 * D_FEAT, INTERNAL), lambda k: (0, 0)),
        ],
        out_specs=pl.BlockSpec((N1, INTERNAL), lambda k: (0, 0)),
        out_shape=jax.ShapeDtypeStruct((N1, INTERNAL), jnp.float32),
        scratch_shapes=[pltpu.VMEM((N1, INTERNAL), jnp.float32)],
        compiler_params=pltpu.CompilerParams(
            dimension_semantics=("arbitrary",),
        ),
    )(dm1, src_feat, dst_feat, W1)


def _tc_layer2_body(dm2_ref, sf2_ref, df2_ref, w2_ref, wc_ref, out_ref):
    agg = jnp.dot(dm2_ref[...], sf2_ref[...],
                  preferred_element_type=jnp.float32)
    w2 = w2_ref[...]
    h = jnp.maximum(
        jnp.dot(agg, w2[:INTERNAL, :], preferred_element_type=jnp.float32)
        + jnp.dot(df2_ref[...], w2[INTERNAL:, :],
                  preferred_element_type=jnp.float32),
        0.0)
    logits = jnp.dot(h, wc_ref[...], preferred_element_type=jnp.float32)
    m = jnp.max(logits, axis=-1, keepdims=True)
    e = jnp.exp(logits - m)
    out_ref[...] = e / jnp.sum(e, axis=-1, keepdims=True)


def _tc_layer2(dm2, src_feat2, dst_feat2, W2, Wc):
    return pl.pallas_call(
        _tc_layer2_body,
        out_shape=jax.ShapeDtypeStruct((B, NUM_CLASSES), jnp.float32),
    )(dm2, src_feat2, dst_feat2, W2, Wc)


def kernel(features, src_nodes, dstsrc2src_l1, dstsrc2dst_l1, dif_mat_l1,
           dstsrc2src_l2, dstsrc2dst_l2, dif_mat_l2, W1, W2, Wc):
    src_feat1, dst_feat1 = _sc_gather_l1(
        features, src_nodes, dstsrc2src_l1, dstsrc2dst_l1)
    h1 = _tc_layer1(dif_mat_l1, src_feat1, dst_feat1, W1)
    src_feat2, dst_feat2 = _sc_gather_l2(h1, dstsrc2src_l2, dstsrc2dst_l2)
    return _tc_layer2(dif_mat_l2, src_feat2, dst_feat2, W2, Wc)


# speedup vs baseline: 1.6823x; 1.0350x over previous
import jax
import jax.numpy as jnp
from jax import lax
from jax.experimental import pallas as pl
from jax.experimental.pallas import tpu as pltpu
from jax.experimental.pallas import tpu_sc as plsc

N_NODES, D_FEAT = 100000, 128
N0, N1, B = 30976, 2816, 256
INTERNAL, NUM_CLASSES = 128, 64

NC, NS = 2, 16
NW = NC * NS
CHUNK = 128
N0_CHUNKS = N0 // CHUNK
N1_CHUNKS = N1 // CHUNK
B_CHUNKS = B // CHUNK

HALF_CHUNKS = N0_CHUNKS // 2
HALF_ROWS = HALF_CHUNKS * CHUNK


def _compose_gather_chunk(features, src_nodes, idx_hbm, out_hbm, base_in,
                          base_out, j_v, idx_v, rows_v, sem, sem2):
    pltpu.sync_copy(idx_hbm.at[pl.ds(base_in, CHUNK)], j_v)
    pltpu.async_copy(src_nodes.at[j_v], idx_v, sem2).wait()
    pltpu.async_copy(features.at[idx_v], rows_v, sem).wait()
    pltpu.sync_copy(rows_v, out_hbm.at[pl.ds(base_out, CHUNK)])


def _sc_gather_half_body(with_dst, features, src_nodes, d2s, d2d, src_out,
                         dst_out, j_v, idx_v, rows_v, sem, sem2):
    wid = lax.axis_index("s") * NC + lax.axis_index("c")

    rounds = -(-HALF_CHUNKS // NW)
    for t in range(rounds):
        cid = wid + t * NW

        @pl.when(cid < HALF_CHUNKS)
        def _():
            base = pl.multiple_of(cid * CHUNK, CHUNK)
            off = (HALF_ROWS if with_dst else 0) + base
            _compose_gather_chunk(features, src_nodes, d2s, src_out,
                                  off, base, j_v, idx_v, rows_v, sem, sem2)

    if with_dst:
        @pl.when(wid < N1_CHUNKS)
        def _():
            base = pl.multiple_of(wid * CHUNK, CHUNK)
            _compose_gather_chunk(features, src_nodes, d2d, dst_out,
                                  base, base, j_v, idx_v, rows_v, sem, sem2)


def _sc_gather_l1_half(features, src_nodes, d2s, d2d, with_dst):
    mesh = plsc.VectorSubcoreMesh(core_axis_name="c", subcore_axis_name="s")
    out_type = [jax.ShapeDtypeStruct((HALF_ROWS, D_FEAT), jnp.float32)]
    if with_dst:
        out_type.append(jax.ShapeDtypeStruct((N1, D_FEAT), jnp.float32))

    def body(*args):
        if with_dst:
            _sc_gather_half_body(True, *args)
        else:
            features_, src_nodes_, d2s_, d2d_, src_out_, *rest = args
            _sc_gather_half_body(False, features_, src_nodes_, d2s_, d2d_,
                                 src_out_, None, *rest)

    return pl.kernel(
        body,
        out_type=out_type,
        mesh=mesh,
        scratch_types=[
            pltpu.VMEM((CHUNK,), jnp.int32),
            pltpu.VMEM((CHUNK,), jnp.int32),
            pltpu.VMEM((CHUNK, D_FEAT), jnp.float32),
            pltpu.SemaphoreType.DMA,
            pltpu.SemaphoreType.DMA,
        ],
    )(features, src_nodes, d2s, d2d)


def _sc_gather_l2_body(h1, d2s, d2d, src_out, dst_out, j_v, rows_v, sem):
    wid = lax.axis_index("s") * NC + lax.axis_index("c")

    def do_chunk(cid, idx_hbm, out_hbm):
        base = pl.multiple_of(cid * CHUNK, CHUNK)
        pltpu.sync_copy(idx_hbm.at[pl.ds(base, CHUNK)], j_v)
        pltpu.async_copy(h1.at[j_v], rows_v, sem).wait()
        pltpu.sync_copy(rows_v, out_hbm.at[pl.ds(base, CHUNK)])

    @pl.when(wid < N1_CHUNKS)
    def _():
        do_chunk(wid, d2s, src_out)

    @pl.when((wid >= N1_CHUNKS) & (wid < N1_CHUNKS + B_CHUNKS))
    def _():
        do_chunk(wid - N1_CHUNKS, d2d, dst_out)


def _sc_gather_l2(h1, d2s, d2d):
    mesh = plsc.VectorSubcoreMesh(core_axis_name="c", subcore_axis_name="s")
    return pl.kernel(
        _sc_gather_l2_body,
        out_type=[
            jax.ShapeDtypeStruct((N1, INTERNAL), jnp.float32),
            jax.ShapeDtypeStruct((B, INTERNAL), jnp.float32),
        ],
        mesh=mesh,
        scratch_types=[
            pltpu.VMEM((CHUNK,), jnp.int32),
            pltpu.VMEM((CHUNK, INTERNAL), jnp.float32),
            pltpu.SemaphoreType.DMA,
        ],
    )(h1, d2s, d2d)


K_BLK = 1408
K_STEPS_HALF = HALF_ROWS // K_BLK


def _tc_half1_body(dm_ref, sf_ref, out_ref, acc_ref):
    k = pl.program_id(0)

    @pl.when(k == 0)
    def _():
        acc_ref[...] = jnp.zeros_like(acc_ref)

    acc_ref[...] += jnp.dot(dm_ref[...], sf_ref[...],
                            preferred_element_type=jnp.float32)

    @pl.when(k == K_STEPS_HALF - 1)
    def _():
        out_ref[...] = acc_ref[...]


def _tc_half1(dm1, src_feat_a):
    return pl.pallas_call(
        _tc_half1_body,
        grid=(K_STEPS_HALF,),
        in_specs=[
            pl.BlockSpec((N1, K_BLK), lambda k: (0, k)),
            pl.BlockSpec((K_BLK, D_FEAT), lambda k: (k, 0)),
        ],
        out_specs=pl.BlockSpec((N1, D_FEAT), lambda k: (0, 0)),
        out_shape=jax.ShapeDtypeStruct((N1, D_FEAT), jnp.float32),
        scratch_shapes=[pltpu.VMEM((N1, D_FEAT), jnp.float32)],
        compiler_params=pltpu.CompilerParams(
            dimension_semantics=("arbitrary",),
        ),
    )(dm1, src_feat_a)


def _tc_half2_body(dm_ref, sf_ref, acc_in_ref, df_ref, w1_ref, out_ref,
                   acc_ref):
    k = pl.program_id(0)

    @pl.when(k == 0)
    def _():
        acc_ref[...] = acc_in_ref[...]

    acc_ref[...] += jnp.dot(dm_ref[...], sf_ref[...],
                            preferred_element_type=jnp.float32)

    @pl.when(k == K_STEPS_HALF - 1)
    def _():
        w1 = w1_ref[...]
        h = (jnp.dot(acc_ref[...], w1[:D_FEAT, :],
                     preferred_element_type=jnp.float32)
             + jnp.dot(df_ref[...], w1[D_FEAT:, :],
                       preferred_element_type=jnp.float32))
        out_ref[...] = jnp.maximum(h, 0.0)


def _tc_half2(dm1, src_feat_b, acc_in, dst_feat, W1):
    return pl.pallas_call(
        _tc_half2_body,
        grid=(K_STEPS_HALF,),
        in_specs=[
            pl.BlockSpec((N1, K_BLK), lambda k: (0, k + K_STEPS_HALF)),
            pl.BlockSpec((K_BLK, D_FEAT), lambda k: (k, 0)),
            pl.BlockSpec((N1, D_FEAT), lambda k: (0, 0)),
            pl.BlockSpec((N1, D_FEAT), lambda k: (0, 0)),
            pl.BlockSpec((2 * D_FEAT, INTERNAL), lambda k: (0, 0)),
        ],
        out_specs=pl.BlockSpec((N1, INTERNAL), lambda k: (0, 0)),
        out_shape=jax.ShapeDtypeStruct((N1, INTERNAL), jnp.float32),
        scratch_shapes=[pltpu.VMEM((N1, INTERNAL), jnp.float32)],
        compiler_params=pltpu.CompilerParams(
            dimension_semantics=("arbitrary",),
        ),
    )(dm1, src_feat_b, acc_in, dst_feat, W1)


def _tc_layer2_body(dm2_ref, sf2_ref, df2_ref, w2_ref, wc_ref, out_ref):
    agg = jnp.dot(dm2_ref[...], sf2_ref[...],
                  preferred_element_type=jnp.float32)
    w2 = w2_ref[...]
    h = jnp.maximum(
        jnp.dot(agg, w2[:INTERNAL, :], preferred_element_type=jnp.float32)
        + jnp.dot(df2_ref[...], w2[INTERNAL:, :],
                  preferred_element_type=jnp.float32),
        0.0)
    logits = jnp.dot(h, wc_ref[...], preferred_element_type=jnp.float32)
    m = jnp.max(logits, axis=-1, keepdims=True)
    e = jnp.exp(logits - m)
    out_ref[...] = e / jnp.sum(e, axis=-1, keepdims=True)


def _tc_layer2(dm2, src_feat2, dst_feat2, W2, Wc):
    return pl.pallas_call(
        _tc_layer2_body,
        out_shape=jax.ShapeDtypeStruct((B, NUM_CLASSES), jnp.float32),
    )(dm2, src_feat2, dst_feat2, W2, Wc)


def kernel(features, src_nodes, dstsrc2src_l1, dstsrc2dst_l1, dif_mat_l1,
           dstsrc2src_l2, dstsrc2dst_l2, dif_mat_l2, W1, W2, Wc):
    (src_feat_a,) = _sc_gather_l1_half(
        features, src_nodes, dstsrc2src_l1, dstsrc2dst_l1, with_dst=False)
    src_feat_b, dst_feat1 = _sc_gather_l1_half(
        features, src_nodes, dstsrc2src_l1, dstsrc2dst_l1, with_dst=True)
    acc = _tc_half1(dif_mat_l1, src_feat_a)
    h1 = _tc_half2(dif_mat_l1, src_feat_b, acc, dst_feat1, W1)
    src_feat2, dst_feat2 = _sc_gather_l2(h1, dstsrc2src_l2, dstsrc2dst_l2)
    return _tc_layer2(dif_mat_l2, src_feat2, dst_feat2, W2, Wc)
